# baseline (speedup 1.0000x reference)
import jax
import jax.numpy as jnp
from jax.experimental import pallas as pl
from jax.experimental.pallas import tpu as pltpu
from jax.sharding import PartitionSpec as P

DIM = 128
HEADS = 4
T = 2
HF = 64
H = W = 32
B = 2
POS = H * W
N = POS * T
CG = DIM // T
CPH = CG // HEADS
KTOP = int(N * 0.25)
ROWS = 512
ITERS = 12


def _ln_cm(x, g, b):
    mu = jnp.mean(x, axis=0, keepdims=True)
    xc = x - mu
    var = jnp.mean(xc * xc, axis=0, keepdims=True)
    return xc * jax.lax.rsqrt(var + 1e-5) * g + b


def _ln_tm(x, g, b):
    mu = jnp.mean(x, axis=1, keepdims=True)
    xc = x - mu
    var = jnp.mean(xc * xc, axis=1, keepdims=True)
    return xc * jax.lax.rsqrt(var + 1e-5) * g + b


def _pad_conv9(y, w_ref, ch):
    yim = y.reshape(H, W, ch)
    zc = jnp.zeros((H, 1, ch), jnp.float32)
    yim = jnp.concatenate([zc, yim, zc], axis=1)
    zr = jnp.zeros((1, W + 2, ch), jnp.float32)
    yim = jnp.concatenate([zr, yim, zr], axis=0)
    acc = yim[0:H, 0:W, :] * w_ref[0:1, :].reshape(1, 1, ch)
    for tap in range(1, 9):
        dy, dx = tap // 3, tap % 3
        acc = acc + yim[dy:dy + H, dx:dx + W, :] * w_ref[tap:tap + 1, :].reshape(1, 1, ch)
    return acc


def _pre_kernel(x_ref, g_ref, b_ref, wmix_ref, w_ref, t_ref, q_ref, k_ref, v_ref):
    xn = _ln_cm(x_ref[...], g_ref[...], b_ref[...])
    y = jax.lax.dot_general(xn, wmix_ref[...], (((0,), (0,)), ((), ())),
                            preferred_element_type=jnp.float32)
    acc = _pad_conv9(y, w_ref, 3 * T * CG).reshape(POS, 3 * T * CG)
    qs = [jnp.concatenate([acc[:, t * CG + h2 * CPH:t * CG + (h2 + 1) * CPH]
                           for t in range(T)], axis=0) for h2 in range(HEADS)]
    vs = [jnp.concatenate([acc[:, (2 * T + t) * CG + h2 * CPH:(2 * T + t) * CG + (h2 + 1) * CPH]
                           for t in range(T)], axis=0) for h2 in range(HEADS)]
    q = jnp.stack(qs)
    v = jnp.stack(vs)
    k = jnp.concatenate([acc[:, (T + t) * CG:(T + 1 + t) * CG].T
                         for t in range(T)], axis=1)
    qn = jnp.maximum(jnp.sqrt(jnp.sum(q * q, axis=1, keepdims=True)), 1e-12)
    kn = jnp.maximum(jnp.sqrt(jnp.sum(k * k, axis=1, keepdims=True)), 1e-12)
    q_ref[...] = ((q / qn) * t_ref[...]).astype(jnp.bfloat16)
    k_ref[...] = (k / kn).astype(jnp.bfloat16)
    v_ref[...] = v.astype(jnp.bfloat16)


def _attn_kernel(q_ref, k_ref, v_ref, o_ref):
    s = jax.lax.dot_general(q_ref[...], k_ref[...], (((1,), (0,)), ((), ())),
                            preferred_element_type=jnp.float32)
    rowmax = jnp.max(s, axis=1, keepdims=True)
    lo = jnp.min(s, axis=1, keepdims=True)
    hi = rowmax
    for _ in range(ITERS):
        mid = 0.5 * (lo + hi)
        cnt = jnp.sum(jnp.where(s >= mid, 1.0, 0.0), axis=1, keepdims=True)
        ge = cnt >= KTOP
        lo = jnp.where(ge, mid, lo)
        hi = jnp.where(ge, hi, mid)
    e = jnp.where(s >= lo, jnp.exp(s), 0.0)
    z = jnp.sum(e, axis=1, keepdims=True)
    ev = jax.lax.dot_general(e.astype(jnp.bfloat16), v_ref[...], (((1,), (0,)), ((), ())),
                             preferred_element_type=jnp.float32)
    o_ref[...] = ev / z


def _post_kernel(a_ref, x_ref, pw_ref, g_ref, b_ref, fw_ref, wf_ref, ow_ref, o_ref):
    a = jnp.concatenate([a_ref[h2, t] for t in range(T) for h2 in range(HEADS)],
                        axis=1)
    y = jax.lax.dot_general(a, pw_ref[...], (((1,), (1,)), ((), ())),
                            preferred_element_type=jnp.float32)
    x1 = x_ref[...].T + y
    xn = _ln_tm(x1, g_ref[...], b_ref[...])
    h0 = jax.lax.dot_general(xn, fw_ref[...], (((1,), (1,)), ((), ())),
                             preferred_element_type=jnp.float32)
    hact = jnp.maximum(_pad_conv9(h0, wf_ref, HF), 0.0).reshape(POS, HF)
    y2 = jax.lax.dot_general(hact, ow_ref[...], (((1,), (1,)), ((), ())),
                             preferred_element_type=jnp.float32)
    o_ref[...] = (x1 + y2).T


def _forward(x_cm, n1w, n1b, wmix, wtap_q, tvec, proj_w, n2w, n2b,
             ffn_in_w, wtap_f, ffn_out_w):
    f32 = jnp.float32
    b_l = x_cm.shape[0]
    bh_l = b_l * HEADS

    qn, kn, v8 = pl.pallas_call(
        _pre_kernel,
        grid=(b_l,),
        out_shape=[jax.ShapeDtypeStruct((b_l, HEADS, N, CPH), jnp.bfloat16),
                   jax.ShapeDtypeStruct((b_l, CG, N), jnp.bfloat16),
                   jax.ShapeDtypeStruct((b_l, HEADS, N, CPH), jnp.bfloat16)],
        in_specs=[pl.BlockSpec((None, DIM, POS), lambda i: (i, 0, 0)),
                  pl.BlockSpec((DIM, 1), lambda i: (0, 0)),
                  pl.BlockSpec((DIM, 1), lambda i: (0, 0)),
                  pl.BlockSpec((DIM, 3 * T * CG), lambda i: (0, 0)),
                  pl.BlockSpec((9, 3 * T * CG), lambda i: (0, 0)),
                  pl.BlockSpec((HEADS, 1, 1), lambda i: (0, 0, 0))],
        out_specs=[pl.BlockSpec((None, HEADS, N, CPH), lambda i: (i, 0, 0, 0)),
                   pl.BlockSpec((None, CG, N), lambda i: (i, 0, 0)),
                   pl.BlockSpec((None, HEADS, N, CPH), lambda i: (i, 0, 0, 0))],
    )(x_cm, n1w, n1b, wmix, wtap_q, tvec)

    qn = qn.reshape(bh_l, N, CPH)
    kn = kn.reshape(bh_l, CPH, N)
    v8 = v8.reshape(bh_l, N, CPH)

    outt = pl.pallas_call(
        _attn_kernel,
        grid=(bh_l, N // ROWS),
        out_shape=jax.ShapeDtypeStruct((bh_l, N, CPH), f32),
        in_specs=[pl.BlockSpec((None, ROWS, CPH), lambda i, j: (i, j, 0)),
                  pl.BlockSpec((None, CPH, N), lambda i, j: (i, 0, 0)),
                  pl.BlockSpec((None, N, CPH), lambda i, j: (i, 0, 0))],
        out_specs=pl.BlockSpec((None, ROWS, CPH), lambda i, j: (i, j, 0)),
        compiler_params=pltpu.CompilerParams(
            dimension_semantics=("arbitrary", "arbitrary")),
    )(qn, kn, v8)

    a5 = outt.reshape(b_l, HEADS, T, POS, CPH)
    out = pl.pallas_call(
        _post_kernel,
        grid=(b_l,),
        out_shape=jax.ShapeDtypeStruct((b_l, DIM, POS), f32),
        in_specs=[pl.BlockSpec((None, HEADS, T, POS, CPH), lambda i: (i, 0, 0, 0, 0)),
                  pl.BlockSpec((None, DIM, POS), lambda i: (i, 0, 0)),
                  pl.BlockSpec((DIM, DIM), lambda i: (0, 0)),
                  pl.BlockSpec((1, DIM), lambda i: (0, 0)),
                  pl.BlockSpec((1, DIM), lambda i: (0, 0)),
                  pl.BlockSpec((HF, DIM), lambda i: (0, 0)),
                  pl.BlockSpec((9, HF), lambda i: (0, 0)),
                  pl.BlockSpec((DIM, HF), lambda i: (0, 0))],
        out_specs=pl.BlockSpec((None, DIM, POS), lambda i: (i, 0, 0)),
    )(a5, x_cm, proj_w, n2w, n2b, ffn_in_w, wtap_f, ffn_out_w)

    return out


def kernel(x, norm1_w, norm1_b, qkv_w, qkv_dw_w, temperature, proj_w,
           norm2_w, norm2_b, ffn_in_w, ffn_dw_w, ffn_out_w):
    f32 = jnp.float32
    x_cm = x.reshape(B, DIM, POS)

    wmix = jnp.einsum('ot,cd->tcod', qkv_w, jnp.eye(CG, dtype=f32)).reshape(DIM, 3 * T * CG)
    wtap_q = jnp.repeat(qkv_dw_w[:, 0].transpose(1, 2, 0).reshape(9, 3 * T), CG, axis=1)
    wtap_f = ffn_dw_w[:, 0].transpose(1, 2, 0).reshape(9, HF)
    tvec = temperature.reshape(HEADS, 1, 1)
    args = (norm1_w.reshape(DIM, 1), norm1_b.reshape(DIM, 1), wmix, wtap_q, tvec,
            proj_w, norm2_w.reshape(1, DIM), norm2_b.reshape(1, DIM),
            ffn_in_w, wtap_f, ffn_out_w)

    if jax.device_count() >= 2:
        mesh = jax.make_mesh((2,), ('d',),
                             axis_types=(jax.sharding.AxisType.Explicit,))
        xs = jax.reshard(x_cm, jax.sharding.NamedSharding(mesh, P('d')))
        args_r = tuple(jax.reshard(a, jax.sharding.NamedSharding(mesh, P()))
                       for a in args)
        reps = tuple(P() for _ in args)
        fwd = jax.shard_map(_forward, mesh=mesh, in_specs=(P('d'),) + reps,
                            out_specs=P('d'), check_vma=False)
        out = fwd(xs, *args_r)
    else:
        out = _forward(x_cm, *args)

    return out.reshape(B, DIM, H, W)


# speedup vs baseline: 2.6601x; 2.6601x over previous
import jax
import jax.numpy as jnp
from jax.experimental import pallas as pl
from jax.experimental.pallas import tpu as pltpu

DIM = 128
HEADS = 4
T = 2
HF = 64
H = W = 32
B = 2
POS = H * W
N = POS * T
CG = DIM // T
CPH = CG // HEADS
BH = B * HEADS
KTOP = int(N * 0.25)
ROWS = 1024
ITERS = 12


def _ln_cm(x, g, b):
    mu = jnp.mean(x, axis=0, keepdims=True)
    xc = x - mu
    var = jnp.mean(xc * xc, axis=0, keepdims=True)
    return xc * jax.lax.rsqrt(var + 1e-5) * g + b


def _ln_tm(x, g, b):
    mu = jnp.mean(x, axis=1, keepdims=True)
    xc = x - mu
    var = jnp.mean(xc * xc, axis=1, keepdims=True)
    return xc * jax.lax.rsqrt(var + 1e-5) * g + b


def _pad_conv9(y, w_ref, ch):
    yim = y.reshape(H, W, ch)
    zc = jnp.zeros((H, 1, ch), jnp.float32)
    yim = jnp.concatenate([zc, yim, zc], axis=1)
    zr = jnp.zeros((1, W + 2, ch), jnp.float32)
    yim = jnp.concatenate([zr, yim, zr], axis=0)
    acc = yim[0:H, 0:W, :] * w_ref[0:1, :].reshape(1, 1, ch)
    for tap in range(1, 9):
        dy, dx = tap // 3, tap % 3
        acc = acc + yim[dy:dy + H, dx:dx + W, :] * w_ref[tap:tap + 1, :].reshape(1, 1, ch)
    return acc


def _pre_kernel(x_ref, g_ref, b_ref, wmix_ref, w_ref, t_ref, q_ref, k_ref, v_ref):
    xn = _ln_cm(x_ref[...], g_ref[...], b_ref[...])
    y = jax.lax.dot_general(xn, wmix_ref[...], (((0,), (0,)), ((), ())),
                            preferred_element_type=jnp.float32)
    acc = _pad_conv9(y, w_ref, 3 * T * CG).reshape(POS, 3 * T * CG)
    qs = [jnp.concatenate([acc[:, t * CG + h2 * CPH:t * CG + (h2 + 1) * CPH]
                           for t in range(T)], axis=0) for h2 in range(HEADS)]
    vs = [jnp.concatenate([acc[:, (2 * T + t) * CG + h2 * CPH:(2 * T + t) * CG + (h2 + 1) * CPH]
                           for t in range(T)], axis=0) for h2 in range(HEADS)]
    q = jnp.stack(qs)
    v = jnp.stack(vs)
    k = jnp.concatenate([acc[:, (T + t) * CG:(T + 1 + t) * CG].T
                         for t in range(T)], axis=1)
    qn = jnp.maximum(jnp.sqrt(jnp.sum(q * q, axis=1, keepdims=True)), 1e-12)
    kn = jnp.maximum(jnp.sqrt(jnp.sum(k * k, axis=1, keepdims=True)), 1e-12)
    q_ref[...] = ((q / qn) * t_ref[...]).astype(jnp.bfloat16)
    k_ref[...] = (k / kn).astype(jnp.bfloat16)
    v_ref[...] = v.astype(jnp.bfloat16)


def _attn_kernel(q_ref, k_ref, v_ref, o_ref):
    s = jax.lax.dot_general(q_ref[...], k_ref[...], (((1,), (0,)), ((), ())),
                            preferred_element_type=jnp.float32)
    rowmax = jnp.max(s, axis=1, keepdims=True)
    lo = jnp.min(s, axis=1, keepdims=True)
    hi = rowmax
    for _ in range(ITERS):
        mid = 0.5 * (lo + hi)
        cnt = jnp.sum(jnp.where(s >= mid, 1.0, 0.0), axis=1, keepdims=True)
        ge = cnt >= KTOP
        lo = jnp.where(ge, mid, lo)
        hi = jnp.where(ge, hi, mid)
    e = jnp.where(s >= lo, jnp.exp(s), 0.0)
    z = jnp.sum(e, axis=1, keepdims=True)
    ev = jax.lax.dot_general(e.astype(jnp.bfloat16), v_ref[...], (((1,), (0,)), ((), ())),
                             preferred_element_type=jnp.float32)
    o_ref[...] = ev / z


def _post_kernel(a_ref, x_ref, pw_ref, g_ref, b_ref, fw_ref, wf_ref, ow_ref, o_ref):
    a = jnp.concatenate([a_ref[h2, t] for t in range(T) for h2 in range(HEADS)],
                        axis=1)
    y = jax.lax.dot_general(a, pw_ref[...], (((1,), (1,)), ((), ())),
                            preferred_element_type=jnp.float32)
    x1 = x_ref[...].T + y
    xn = _ln_tm(x1, g_ref[...], b_ref[...])
    h0 = jax.lax.dot_general(xn, fw_ref[...], (((1,), (1,)), ((), ())),
                             preferred_element_type=jnp.float32)
    hact = jnp.maximum(_pad_conv9(h0, wf_ref, HF), 0.0).reshape(POS, HF)
    y2 = jax.lax.dot_general(hact, ow_ref[...], (((1,), (1,)), ((), ())),
                             preferred_element_type=jnp.float32)
    o_ref[...] = (x1 + y2).T


def kernel(x, norm1_w, norm1_b, qkv_w, qkv_dw_w, temperature, proj_w,
           norm2_w, norm2_b, ffn_in_w, ffn_dw_w, ffn_out_w):
    f32 = jnp.float32
    x_cm = x.reshape(B, DIM, POS)

    wmix = jnp.einsum('ot,cd->tcod', qkv_w, jnp.eye(CG, dtype=f32)).reshape(DIM, 3 * T * CG)
    wtap_q = jnp.repeat(qkv_dw_w[:, 0].transpose(1, 2, 0).reshape(9, 3 * T), CG, axis=1)
    wtap_f = ffn_dw_w[:, 0].transpose(1, 2, 0).reshape(9, HF)
    tvec = temperature.reshape(HEADS, 1, 1)

    qn, kn, v8 = pl.pallas_call(
        _pre_kernel,
        grid=(B,),
        out_shape=[jax.ShapeDtypeStruct((B, HEADS, N, CPH), jnp.bfloat16),
                   jax.ShapeDtypeStruct((B, CG, N), jnp.bfloat16),
                   jax.ShapeDtypeStruct((B, HEADS, N, CPH), jnp.bfloat16)],
        in_specs=[pl.BlockSpec((None, DIM, POS), lambda i: (i, 0, 0)),
                  pl.BlockSpec((DIM, 1), lambda i: (0, 0)),
                  pl.BlockSpec((DIM, 1), lambda i: (0, 0)),
                  pl.BlockSpec((DIM, 3 * T * CG), lambda i: (0, 0)),
                  pl.BlockSpec((9, 3 * T * CG), lambda i: (0, 0)),
                  pl.BlockSpec((HEADS, 1, 1), lambda i: (0, 0, 0))],
        out_specs=[pl.BlockSpec((None, HEADS, N, CPH), lambda i: (i, 0, 0, 0)),
                   pl.BlockSpec((None, CG, N), lambda i: (i, 0, 0)),
                   pl.BlockSpec((None, HEADS, N, CPH), lambda i: (i, 0, 0, 0))],
    )(x_cm, norm1_w.reshape(DIM, 1), norm1_b.reshape(DIM, 1), wmix, wtap_q, tvec)

    qn = qn.reshape(BH, N, CPH)
    kn = kn.reshape(BH, CPH, N)
    v8 = v8.reshape(BH, N, CPH)

    outt = pl.pallas_call(
        _attn_kernel,
        grid=(BH, N // ROWS),
        out_shape=jax.ShapeDtypeStruct((BH, N, CPH), f32),
        in_specs=[pl.BlockSpec((None, ROWS, CPH), lambda i, j: (i, j, 0)),
                  pl.BlockSpec((None, CPH, N), lambda i, j: (i, 0, 0)),
                  pl.BlockSpec((None, N, CPH), lambda i, j: (i, 0, 0))],
        out_specs=pl.BlockSpec((None, ROWS, CPH), lambda i, j: (i, j, 0)),
        compiler_params=pltpu.CompilerParams(
            dimension_semantics=("arbitrary", "arbitrary")),
    )(qn, kn, v8)

    a5 = outt.reshape(B, HEADS, T, POS, CPH)
    out = pl.pallas_call(
        _post_kernel,
        grid=(B,),
        out_shape=jax.ShapeDtypeStruct((B, DIM, POS), f32),
        in_specs=[pl.BlockSpec((None, HEADS, T, POS, CPH), lambda i: (i, 0, 0, 0, 0)),
                  pl.BlockSpec((None, DIM, POS), lambda i: (i, 0, 0)),
                  pl.BlockSpec((DIM, DIM), lambda i: (0, 0)),
                  pl.BlockSpec((1, DIM), lambda i: (0, 0)),
                  pl.BlockSpec((1, DIM), lambda i: (0, 0)),
                  pl.BlockSpec((HF, DIM), lambda i: (0, 0)),
                  pl.BlockSpec((9, HF), lambda i: (0, 0)),
                  pl.BlockSpec((DIM, HF), lambda i: (0, 0))],
        out_specs=pl.BlockSpec((None, DIM, POS), lambda i: (i, 0, 0)),
    )(a5, x_cm, proj_w, norm2_w.reshape(1, DIM), norm2_b.reshape(1, DIM),
      ffn_in_w, wtap_f, ffn_out_w)

    return out.reshape(B, DIM, H, W)


# speedup vs baseline: 2.7570x; 1.0364x over previous
import jax
import jax.numpy as jnp
from jax.experimental import pallas as pl
from jax.experimental.pallas import tpu as pltpu

DIM = 128
HEADS = 4
T = 2
HF = 64
H = W = 32
B = 2
POS = H * W
N = POS * T
CG = DIM // T
CPH = CG // HEADS
BH = B * HEADS
KTOP = int(N * 0.25)
ROWS = 1024
ITERS = 12


def _ln_cm(x, g, b):
    mu = jnp.mean(x, axis=0, keepdims=True)
    xc = x - mu
    var = jnp.mean(xc * xc, axis=0, keepdims=True)
    return xc * jax.lax.rsqrt(var + 1e-5) * g + b


def _ln_tm(x, g, b):
    mu = jnp.mean(x, axis=1, keepdims=True)
    xc = x - mu
    var = jnp.mean(xc * xc, axis=1, keepdims=True)
    return xc * jax.lax.rsqrt(var + 1e-5) * g + b


def _pad_conv9(y, w_ref, ch):
    yim = y.reshape(H, W, ch)
    zc = jnp.zeros((H, 1, ch), jnp.float32)
    yim = jnp.concatenate([zc, yim, zc], axis=1)
    zr = jnp.zeros((1, W + 2, ch), jnp.float32)
    yim = jnp.concatenate([zr, yim, zr], axis=0)
    acc = yim[0:H, 0:W, :] * w_ref[0:1, :].reshape(1, 1, ch)
    for tap in range(1, 9):
        dy, dx = tap // 3, tap % 3
        acc = acc + yim[dy:dy + H, dx:dx + W, :] * w_ref[tap:tap + 1, :].reshape(1, 1, ch)
    return acc


def _pre_kernel(x_ref, g_ref, b_ref, wmix_ref, w_ref, t_ref, q_ref, k_ref, v_ref):
    xn = _ln_cm(x_ref[...], g_ref[...], b_ref[...])
    y = jax.lax.dot_general(xn, wmix_ref[...], (((0,), (0,)), ((), ())),
                            preferred_element_type=jnp.float32)
    acc = _pad_conv9(y, w_ref, 3 * T * CG).reshape(POS, 3 * T * CG)
    qs = [jnp.concatenate([acc[:, t * CG + h2 * CPH:t * CG + (h2 + 1) * CPH]
                           for t in range(T)], axis=0) for h2 in range(HEADS)]
    vs = [jnp.concatenate([acc[:, (2 * T + t) * CG + h2 * CPH:(2 * T + t) * CG + (h2 + 1) * CPH]
                           for t in range(T)], axis=0) for h2 in range(HEADS)]
    q = jnp.stack(qs)
    v = jnp.stack(vs)
    k = jnp.concatenate([acc[:, (T + t) * CG:(T + 1 + t) * CG].T
                         for t in range(T)], axis=1)
    qn = jnp.maximum(jnp.sqrt(jnp.sum(q * q, axis=1, keepdims=True)), 1e-12)
    kn = jnp.maximum(jnp.sqrt(jnp.sum(k * k, axis=1, keepdims=True)), 1e-12)
    q_ref[...] = ((q / qn) * t_ref[...]).astype(jnp.bfloat16)
    k_ref[...] = (k / kn).astype(jnp.bfloat16)
    v_ref[...] = v.astype(jnp.bfloat16)


def _attn_half(q, k, v):
    s = jax.lax.dot_general(q, k, (((1,), (0,)), ((), ())),
                            preferred_element_type=jnp.float32)
    rowmax = jnp.max(s, axis=1, keepdims=True)
    lo = jnp.min(s, axis=1, keepdims=True)
    hi = rowmax
    for _ in range(ITERS):
        mid = 0.5 * (lo + hi)
        cnt = jnp.sum(jnp.where(s >= mid, 1.0, 0.0), axis=1, keepdims=True)
        ge = cnt >= KTOP
        lo = jnp.where(ge, mid, lo)
        hi = jnp.where(ge, hi, mid)
    e = jnp.where(s >= lo, jnp.exp(s), 0.0)
    z = jnp.sum(e, axis=1, keepdims=True)
    ev = jax.lax.dot_general(e.astype(jnp.bfloat16), v, (((1,), (0,)), ((), ())),
                             preferred_element_type=jnp.float32)
    return ev / z


def _attn_kernel(q_ref, k_ref, v_ref, o_ref):
    q = q_ref[...]
    k = k_ref[...]
    v = v_ref[...]
    hr = ROWS // 2
    o_ref[0:hr, :] = _attn_half(q[0:hr], k, v)
    o_ref[hr:ROWS, :] = _attn_half(q[hr:ROWS], k, v)


def _post_kernel(a_ref, x_ref, pw_ref, g_ref, b_ref, fw_ref, wf_ref, ow_ref, o_ref):
    a = jnp.concatenate([a_ref[h2, t] for t in range(T) for h2 in range(HEADS)],
                        axis=1)
    y = jax.lax.dot_general(a, pw_ref[...], (((1,), (1,)), ((), ())),
                            preferred_element_type=jnp.float32)
    x1 = x_ref[...].T + y
    xn = _ln_tm(x1, g_ref[...], b_ref[...])
    h0 = jax.lax.dot_general(xn, fw_ref[...], (((1,), (1,)), ((), ())),
                             preferred_element_type=jnp.float32)
    hact = jnp.maximum(_pad_conv9(h0, wf_ref, HF), 0.0).reshape(POS, HF)
    y2 = jax.lax.dot_general(hact, ow_ref[...], (((1,), (1,)), ((), ())),
                             preferred_element_type=jnp.float32)
    o_ref[...] = (x1 + y2).T


def kernel(x, norm1_w, norm1_b, qkv_w, qkv_dw_w, temperature, proj_w,
           norm2_w, norm2_b, ffn_in_w, ffn_dw_w, ffn_out_w):
    f32 = jnp.float32
    x_cm = x.reshape(B, DIM, POS)

    wmix = jnp.einsum('ot,cd->tcod', qkv_w, jnp.eye(CG, dtype=f32)).reshape(DIM, 3 * T * CG)
    wtap_q = jnp.repeat(qkv_dw_w[:, 0].transpose(1, 2, 0).reshape(9, 3 * T), CG, axis=1)
    wtap_f = ffn_dw_w[:, 0].transpose(1, 2, 0).reshape(9, HF)
    tvec = temperature.reshape(HEADS, 1, 1)

    qn, kn, v8 = pl.pallas_call(
        _pre_kernel,
        grid=(B,),
        out_shape=[jax.ShapeDtypeStruct((B, HEADS, N, CPH), jnp.bfloat16),
                   jax.ShapeDtypeStruct((B, CG, N), jnp.bfloat16),
                   jax.ShapeDtypeStruct((B, HEADS, N, CPH), jnp.bfloat16)],
        in_specs=[pl.BlockSpec((None, DIM, POS), lambda i: (i, 0, 0)),
                  pl.BlockSpec((DIM, 1), lambda i: (0, 0)),
                  pl.BlockSpec((DIM, 1), lambda i: (0, 0)),
                  pl.BlockSpec((DIM, 3 * T * CG), lambda i: (0, 0)),
                  pl.BlockSpec((9, 3 * T * CG), lambda i: (0, 0)),
                  pl.BlockSpec((HEADS, 1, 1), lambda i: (0, 0, 0))],
        out_specs=[pl.BlockSpec((None, HEADS, N, CPH), lambda i: (i, 0, 0, 0)),
                   pl.BlockSpec((None, CG, N), lambda i: (i, 0, 0)),
                   pl.BlockSpec((None, HEADS, N, CPH), lambda i: (i, 0, 0, 0))],
    )(x_cm, norm1_w.reshape(DIM, 1), norm1_b.reshape(DIM, 1), wmix, wtap_q, tvec)

    qn = qn.reshape(BH, N, CPH)
    kn = kn.reshape(BH, CPH, N)
    v8 = v8.reshape(BH, N, CPH)

    outt = pl.pallas_call(
        _attn_kernel,
        grid=(BH, N // ROWS),
        out_shape=jax.ShapeDtypeStruct((BH, N, CPH), f32),
        in_specs=[pl.BlockSpec((None, ROWS, CPH), lambda i, j: (i, j, 0)),
                  pl.BlockSpec((None, CPH, N), lambda i, j: (i, 0, 0)),
                  pl.BlockSpec((None, N, CPH), lambda i, j: (i, 0, 0))],
        out_specs=pl.BlockSpec((None, ROWS, CPH), lambda i, j: (i, j, 0)),
        compiler_params=pltpu.CompilerParams(
            dimension_semantics=("arbitrary", "arbitrary")),
    )(qn, kn, v8)

    a5 = outt.reshape(B, HEADS, T, POS, CPH)
    out = pl.pallas_call(
        _post_kernel,
        grid=(B,),
        out_shape=jax.ShapeDtypeStruct((B, DIM, POS), f32),
        in_specs=[pl.BlockSpec((None, HEADS, T, POS, CPH), lambda i: (i, 0, 0, 0, 0)),
                  pl.BlockSpec((None, DIM, POS), lambda i: (i, 0, 0)),
                  pl.BlockSpec((DIM, DIM), lambda i: (0, 0)),
                  pl.BlockSpec((1, DIM), lambda i: (0, 0)),
                  pl.BlockSpec((1, DIM), lambda i: (0, 0)),
                  pl.BlockSpec((HF, DIM), lambda i: (0, 0)),
                  pl.BlockSpec((9, HF), lambda i: (0, 0)),
                  pl.BlockSpec((DIM, HF), lambda i: (0, 0))],
        out_specs=pl.BlockSpec((None, DIM, POS), lambda i: (i, 0, 0)),
    )(a5, x_cm, proj_w, norm2_w.reshape(1, DIM), norm2_b.reshape(1, DIM),
      ffn_in_w, wtap_f, ffn_out_w)

    return out.reshape(B, DIM, H, W)
